# baseline (speedup 1.0000x reference)
import functools

import jax
import jax.numpy as jnp
from jax import lax
from jax.experimental import pallas as pl
from jax.experimental.pallas import tpu as pltpu
from jax.experimental.pallas import tpu_sc as plsc

D_MODEL = 64
SCALE = 8.0
LANES = 16
BBLK = 128
NG = 4
NO = 2


@functools.lru_cache(maxsize=None)
def _build(batch, hist, vocab):
    info = plsc.get_sparse_core_info()
    nw = info.num_cores * info.num_subcores
    assert batch == nw * BBLK and hist % NG == 0

    mesh = plsc.VectorSubcoreMesh(core_axis_name="c", subcore_axis_name="s")

    @functools.partial(
        pl.kernel,
        mesh=mesh,
        out_type=jax.ShapeDtypeStruct((hist, D_MODEL, batch), jnp.float32),
        scratch_types=[
            pltpu.VMEM((hist, BBLK), jnp.int32),
            pltpu.VMEM((NG, BBLK), jnp.int32),
            pltpu.VMEM((BBLK, 2 * D_MODEL), jnp.float32),
            pltpu.VMEM((BBLK, 2 * D_MODEL), jnp.float32),
            pltpu.VMEM((BBLK, 2 * D_MODEL), jnp.float32),
            pltpu.VMEM((BBLK, 2 * D_MODEL), jnp.float32),
            pltpu.VMEM((D_MODEL, BBLK), jnp.float32),
            pltpu.VMEM((D_MODEL, BBLK), jnp.float32),
            pltpu.SemaphoreType.DMA,
            pltpu.SemaphoreType.DMA,
        ],
        compiler_params=pltpu.CompilerParams(use_tc_tiling_on_sc=True,
                                             needs_layout_passes=False),
    )
    def k(tablep_hbm, xt_hbm, out_hbm, idx_v, pring, g0, g1, g2, g3,
          o0, o1, gsem, wsem):
        gbufs = [g0, g1, g2, g3]
        obufs = [o0, o1]
        wid = lax.axis_index("s") * info.num_cores + lax.axis_index("c")
        bbase = wid * BBLK
        pltpu.sync_copy(xt_hbm.at[pl.ds(0, hist), pl.ds(bbase, BBLK)], idx_v)

        iota16 = lax.iota(jnp.int32, LANES)

        def prep_and_gather(h, gslot):
            for sl in range(BBLK // LANES):
                s = pl.ds(sl * LANES, LANES)
                pring[gslot, s] = lax.shift_right_logical(idx_v[h, s], 1)
            pltpu.async_copy(tablep_hbm.at[pring.at[gslot]], gbufs[gslot],
                             gsem)

        def wait_gather(gslot):
            pltpu.make_async_copy(
                tablep_hbm.at[pring.at[gslot]], gbufs[gslot], gsem).wait()

        def start_write(h, oslot):
            pltpu.async_copy(obufs[oslot],
                             out_hbm.at[h, :, pl.ds(bbase, BBLK)], wsem)

        def wait_write(h, oslot):
            pltpu.make_async_copy(
                obufs[oslot], out_hbm.at[h, :, pl.ds(bbase, BBLK)],
                wsem).wait()

        def transpose_scale(h, gslot, oslot):
            gbuf = gbufs[gslot]
            obuf = obufs[oslot]
            for lg in range(BBLK // LANES):
                s = pl.ds(lg * LANES, LANES)
                rowi = iota16 + (lg * LANES)
                base = (idx_v[h, s] & 1) * D_MODEL

                def d_body(d, carry, rowi=rowi, base=base, s=s,
                           gbuf=gbuf, obuf=obuf):
                    vals = plsc.load_gather(gbuf, [rowi, base + d])
                    obuf[d, s] = vals * SCALE
                    return carry

                lax.fori_loop(0, D_MODEL, d_body, 0, unroll=4)

        def slot_work(h, gslot, oslot, first_write, do_gather):
            if not first_write:
                wait_write(h - NO, oslot)
            wait_gather(gslot)
            transpose_scale(h, gslot, oslot)
            start_write(h, oslot)
            if do_gather:
                prep_and_gather(h + NG, gslot)

        for j in range(NG):
            prep_and_gather(j, j)
        for j in range(NG):
            slot_work(j, j, j % NO, j < NO, True)

        def outer(o, carry):
            h = NG * o
            for j in range(NG):
                slot_work(h + j, j, j % NO, False, True)
            return carry

        lax.fori_loop(1, hist // NG - 1, outer, 0)

        for j in range(NG):
            h = hist - NG + j
            slot_work(h, j, h % NO, False, False)
        wait_write(hist - 2, (hist - 2) % NO)
        wait_write(hist - 1, (hist - 1) % NO)

    return k


def kernel(x, table):
    b, h = x.shape
    vocab = table.shape[0]
    tablep = table.reshape(vocab // 2, 2 * D_MODEL)
    out_t = _build(b, h, vocab)(tablep, x.T.astype(jnp.int32))
    return out_t.transpose(2, 0, 1)


# speedup vs baseline: 1.5924x; 1.5924x over previous
import functools

import jax
import jax.numpy as jnp
from jax import lax
from jax.experimental import pallas as pl
from jax.experimental.pallas import tpu as pltpu
from jax.experimental.pallas import tpu_sc as plsc

D_MODEL = 64
GROUPS = (104, 96)
ROWS_PER_CHUNK = 2
NBUF = 4
LOOK = 3
SCALE = 8.0
LANES = 16


@functools.lru_cache(maxsize=None)
def _build(batch, hist, vocab):
    info = plsc.get_sparse_core_info()
    nw = info.num_cores * info.num_subcores
    b_per_w = batch // nw
    n_chunks = b_per_w // ROWS_PER_CHUNK

    mesh = plsc.VectorSubcoreMesh(core_axis_name="c", subcore_axis_name="s")

    @functools.partial(
        pl.kernel,
        mesh=mesh,
        out_type=jax.ShapeDtypeStruct((batch, hist, D_MODEL), jnp.float32),
        scratch_types=[
            pltpu.VMEM((b_per_w, hist), jnp.int32),
            pltpu.VMEM((ROWS_PER_CHUNK, hist, D_MODEL), jnp.float32),
            pltpu.VMEM((ROWS_PER_CHUNK, hist, D_MODEL), jnp.float32),
            pltpu.VMEM((ROWS_PER_CHUNK, hist, D_MODEL), jnp.float32),
            pltpu.VMEM((ROWS_PER_CHUNK, hist, D_MODEL), jnp.float32),
            pltpu.SemaphoreType.DMA,
            pltpu.SemaphoreType.DMA,
        ],
        compiler_params=pltpu.CompilerParams(use_tc_tiling_on_sc=False),
    )
    def k(table_hbm, x_hbm, out_hbm, idx_v, b0, b1, b2, b3, gsem, ssem):
        bufs = [b0, b1, b2, b3]
        wid = lax.axis_index("s") * info.num_cores + lax.axis_index("c")
        bbase = wid * b_per_w
        pltpu.sync_copy(x_hbm.at[pl.ds(bbase, b_per_w)], idx_v)

        ghandles = {}
        shandles = {}

        def start_gathers(c):
            p = c % NBUF
            hs = []
            for i in range(ROWS_PER_CHUNK):
                off = 0
                for g in GROUPS:
                    hs.append(pltpu.async_copy(
                        table_hbm.at[idx_v.at[c * ROWS_PER_CHUNK + i,
                                              pl.ds(off, g)]],
                        bufs[p].at[i, pl.ds(off, g)],
                        gsem))
                    off += g
            ghandles[c] = hs

        def scale_chunk(p):
            buf = bufs[p]
            for i in range(ROWS_PER_CHUNK):
                def row_body(r, carry, i=i):
                    for q in range(D_MODEL // LANES):
                        sl = pl.ds(q * LANES, LANES)
                        buf[i, r, sl] = buf[i, r, sl] * SCALE
                    return carry

                lax.fori_loop(0, hist, row_body, 0, unroll=4)

        for c in range(LOOK):
            start_gathers(c)
        for c in range(n_chunks):
            p = c % NBUF
            for h in ghandles.pop(c):
                h.wait()
            scale_chunk(p)
            shandles[c] = pltpu.async_copy(
                bufs[p],
                out_hbm.at[pl.ds(bbase + c * ROWS_PER_CHUNK, ROWS_PER_CHUNK)],
                ssem)
            nxt = c + LOOK
            if nxt < n_chunks:
                prev_user = nxt - NBUF
                if prev_user >= 0:
                    shandles.pop(prev_user).wait()
                start_gathers(nxt)
        for c in sorted(shandles):
            shandles.pop(c).wait()

    return k


def kernel(x, table):
    b, h = x.shape
    return _build(b, h, table.shape[0])(table, x.astype(jnp.int32))
